# baseline (speedup 1.0000x reference)
import functools

import jax
import jax.numpy as jnp
from jax import lax
from jax.experimental import pallas as pl
from jax.experimental.pallas import tpu as pltpu
from jax.experimental.pallas import tpu_sc as plsc

_N = 10000
_E = 320000
_DF = 128
_H = 32
_NMP = 5

_NC = 2
_NS = 16
_NW = _NC * _NS
_CHUNK = 128
_EPT = _E // _NW
_NFULL = _EPT // _CHUNK
_TAIL = _EPT - _NFULL * _CHUNK
_NP = 10240
_RPT = _NP // _NS
_NBUF = 8
_NGRP = 9

_mesh = plsc.VectorSubcoreMesh(core_axis_name="c", subcore_axis_name="s")
_sc_params = pltpu.CompilerParams(use_tc_tiling_on_sc=False)


def _zero_shared_rows(zbuf, s_sh, sid):
    def zb(i, carry):
        zbuf[i, 0:16] = jnp.zeros((16,), jnp.float32)
        zbuf[i, 16:32] = jnp.zeros((16,), jnp.float32)
        return carry
    lax.fori_loop(0, _CHUNK, zb, 0)
    for k in range(_RPT // _CHUNK):
        pltpu.sync_copy(zbuf, s_sh.at[pl.ds(sid * _RPT + k * _CHUNK, _CHUNK)])


def _sc_body(src_of, ridx, gbuf, s_sh, gsems, ssems):
    def sdst(j):
        return s_sh.at[ridx.at[pl.ds(j * _CHUNK, _CHUNK)]]

    for b in range(_NBUF):
        pltpu.async_copy(src_of(b), gbuf.at[b], gsems[b])

    def outer(g, carry):
        for b in range(_NBUF):
            j = g * _NBUF + b
            pltpu.make_async_copy(src_of(j), gbuf.at[b], gsems[b]).wait()
            pltpu.async_copy(gbuf.at[b], sdst(j), ssems[b], add=True)
            pb = (b - 2) % _NBUF
            c = j - 2

            @pl.when(jnp.logical_and(c >= 0, c + _NBUF < _NFULL))
            def _():
                pltpu.make_async_copy(gbuf.at[pb], sdst(c), ssems[pb]).wait()
                pltpu.async_copy(src_of(c + _NBUF), gbuf.at[pb], gsems[pb])
        return carry
    lax.fori_loop(0, _NGRP, outer, 0)

    last = _NGRP * _NBUF
    for j in range(last - 2, last):
        pltpu.make_async_copy(gbuf.at[j % _NBUF], sdst(j),
                              ssems[j % _NBUF]).wait()
    for j in range(last, _NFULL):
        b = j % _NBUF
        pltpu.make_async_copy(src_of(j), gbuf.at[b], gsems[b]).wait()
        pltpu.sync_copy(gbuf.at[b], sdst(j), add=True)


def _sc_tail(src_tail, ridx, gbuf, s_sh, sem):
    tb = _NFULL * _CHUNK
    pltpu.async_copy(src_tail, gbuf.at[2, pl.ds(0, _TAIL)], sem).wait()
    pltpu.sync_copy(gbuf.at[2, pl.ds(0, _TAIL)],
                    s_sh.at[ridx.at[pl.ds(tb, _TAIL)]], add=True)


def _drain_shared(s_sh, out, cid, sid):
    pltpu.sync_copy(
        s_sh.at[pl.ds(sid * _RPT, _RPT)],
        out.at[cid, pl.ds(sid * _RPT, _RPT)],
    )


_sc_scratch = [
    pltpu.VMEM((_EPT,), jnp.int32),
    pltpu.VMEM((_NBUF, _CHUNK, _H), jnp.float32),
    pltpu.VMEM((_CHUNK, _H), jnp.float32),
    pltpu.VMEM_SHARED((_NP, _H), jnp.float32),
] + [pltpu.SemaphoreType.DMA] * (2 * _NBUF)


@functools.partial(
    pl.kernel,
    mesh=_mesh,
    out_type=jax.ShapeDtypeStruct((_NC, _NP, _H), jnp.float32),
    scratch_types=[pltpu.VMEM((_EPT,), jnp.int32)] + _sc_scratch,
    compiler_params=_sc_params,
)
def _sc_gather_scatter(hn, send, recv, out, sidx, ridx, gbuf, zbuf, s_sh,
                       *sems):
    cid = lax.axis_index("c")
    sid = lax.axis_index("s")
    wid = cid * _NS + sid
    base = wid * _EPT
    pltpu.sync_copy(send.at[pl.ds(base, _EPT)], sidx)
    pltpu.sync_copy(recv.at[pl.ds(base, _EPT)], ridx)
    _zero_shared_rows(zbuf, s_sh, sid)
    plsc.subcore_barrier()

    def src_of(j):
        return hn.at[sidx.at[pl.ds(j * _CHUNK, _CHUNK)]]

    _sc_body(src_of, ridx, gbuf, s_sh, sems[:_NBUF], sems[_NBUF:])
    _sc_tail(hn.at[sidx.at[pl.ds(_NFULL * _CHUNK, _TAIL)]],
             ridx, gbuf, s_sh, sems[0])

    plsc.subcore_barrier()
    _drain_shared(s_sh, out, cid, sid)


@functools.partial(
    pl.kernel,
    mesh=_mesh,
    out_type=jax.ShapeDtypeStruct((_NC, _NP, _H), jnp.float32),
    scratch_types=_sc_scratch,
    compiler_params=_sc_params,
)
def _sc_segment_sum(vals, recv, out, ridx, gbuf, zbuf, s_sh, *sems):
    cid = lax.axis_index("c")
    sid = lax.axis_index("s")
    wid = cid * _NS + sid
    base = wid * _EPT
    pltpu.sync_copy(recv.at[pl.ds(base, _EPT)], ridx)
    _zero_shared_rows(zbuf, s_sh, sid)
    plsc.subcore_barrier()

    def src_of(j):
        return vals.at[pl.ds(base + j * _CHUNK, _CHUNK)]

    _sc_body(src_of, ridx, gbuf, s_sh, sems[:_NBUF], sems[_NBUF:])
    _sc_tail(vals.at[pl.ds(base + _NFULL * _CHUNK, _TAIL)],
             ridx, gbuf, s_sh, sems[0])

    plsc.subcore_barrier()
    _drain_shared(s_sh, out, cid, sid)


def _full(shape):
    return pl.BlockSpec(shape, lambda i: (0,) * len(shape))


def _mlp2_kernel(x_ref, w1_ref, b1_ref, w2_ref, b2_ref, o_ref):
    y = jnp.dot(x_ref[...], w1_ref[...], preferred_element_type=jnp.float32)
    y = jnp.maximum(y + b1_ref[...], 0.0)
    o_ref[...] = jnp.dot(y, w2_ref[...], preferred_element_type=jnp.float32) + b2_ref[...]


def _mlp2(x, w1, b1, w2, b2, rows_per_block, out_rows=None):
    rows, din = x.shape
    dmid = w1.shape[1]
    dout = w2.shape[1]
    out_rows = rows if out_rows is None else out_rows
    grid = out_rows // rows_per_block
    return pl.pallas_call(
        _mlp2_kernel,
        grid=(grid,),
        in_specs=[
            pl.BlockSpec((rows_per_block, din), lambda i: (i, 0)),
            _full((din, dmid)), _full((1, dmid)),
            _full((dmid, dout)), _full((1, dout)),
        ],
        out_specs=pl.BlockSpec((rows_per_block, dout), lambda i: (i, 0)),
        out_shape=jax.ShapeDtypeStruct((out_rows, dout), jnp.float32),
    )(x, w1, b1.reshape(1, -1), w2, b2.reshape(1, -1))


def _edge_enc_kernel(c0_ref, c1_ref, c2_ref, c3_ref, kcat_ref, b1_ref,
                     w2k_ref, b2_ref, o_ref):
    x = jnp.concatenate(
        [c0_ref[...], c1_ref[...], c2_ref[...], c3_ref[...]], axis=1)
    y = jnp.dot(x, kcat_ref[...], preferred_element_type=jnp.float32)
    y = jnp.maximum(y + b1_ref[...], 0.0)
    for q in range(8):
        z = jnp.dot(y[:, 256 * q:256 * (q + 1)], w2k_ref[...],
                    preferred_element_type=jnp.float32) + b2_ref[...]
        o_ref[1000 * q:1000 * (q + 1), :] = z


def _edge_encode(cols, we0, be0, w2c, b2c):
    f32 = jnp.float32
    eye32 = jnp.eye(32, dtype=f32)
    kcat = jnp.concatenate(
        [jnp.kron(eye32, we0[k].reshape(1, -1)) for k in range(4)])
    b1k = jnp.tile(be0, 32)
    w2k = jnp.kron(jnp.eye(4, dtype=f32), w2c)
    b2k = jnp.tile(b2c, 4)
    rb = 1000
    out = pl.pallas_call(
        _edge_enc_kernel,
        grid=(_N // rb,),
        in_specs=[pl.BlockSpec((rb, _H), lambda i: (i, 0))] * 4
        + [_full((128, 2048)), _full((1, 2048)),
           _full((256, 128)), _full((1, 128))],
        out_specs=pl.BlockSpec((8 * rb, 128), lambda i: (i, 0)),
        out_shape=jax.ShapeDtypeStruct((8 * _N, 128), jnp.float32),
    )(*cols, kcat, b1k.reshape(1, -1), w2k, b2k.reshape(1, -1))
    return out.reshape(_E, _H)


_NPP = _NP // 4


def _upd_p_kernel(hp_ref, sp_ref0, sp_ref1, ae_ref0, ae_ref1,
                  wmt_ref, w0a_ref, w0b_ref, b0_ref, w1_ref, b1_ref,
                  wnode_ref, mones_ref, lns_ref, lnb_ref, o_ref):
    h = hp_ref[...]
    s = sp_ref0[0] + sp_ref1[0]
    agg = (jnp.dot(s, wmt_ref[...], preferred_element_type=jnp.float32)
           + ae_ref0[0] + ae_ref1[0])
    t = (jnp.dot(h, w0a_ref[...], preferred_element_type=jnp.float32)
         + jnp.dot(agg, w0b_ref[...], preferred_element_type=jnp.float32)
         + b0_ref[...])
    t = jnp.maximum(t, 0.0)
    no = jnp.dot(t, w1_ref[...], preferred_element_type=jnp.float32) + b1_ref[...]
    r = jnp.dot(h, wnode_ref[...], preferred_element_type=jnp.float32) + no
    mu = jnp.dot(r, mones_ref[...], preferred_element_type=jnp.float32)
    d = r - mu
    var = jnp.dot(d * d, mones_ref[...], preferred_element_type=jnp.float32)
    o_ref[...] = d * lax.rsqrt(var + 1e-6) * lns_ref[...] + lnb_ref[...]


def _update_p(hp, s_p, agge_p, pw):
    row = lambda i: (i, 0)
    return pl.pallas_call(
        _upd_p_kernel,
        grid=(1,),
        in_specs=[
            pl.BlockSpec((_NPP, 128), row),
            pl.BlockSpec((1, _NPP, 128), lambda i: (0, i, 0)),
            pl.BlockSpec((1, _NPP, 128), lambda i: (1, i, 0)),
            pl.BlockSpec((1, _NPP, 128), lambda i: (0, i, 0)),
            pl.BlockSpec((1, _NPP, 128), lambda i: (1, i, 0)),
        ] + [_full((128, 128))] * 3 + [_full((1, 128))]
        + [_full((128, 128)), _full((1, 128))]
        + [_full((128, 128))] * 2 + [_full((1, 128))] * 2,
        out_specs=pl.BlockSpec((_NPP, 128), row),
        out_shape=jax.ShapeDtypeStruct((_NPP, 128), jnp.float32),
    )(hp, s_p, s_p, agge_p, agge_p, *pw)


def _dec_p_kernel(hp_ref, w1_ref, b1_ref, w2_ref, b2_ref, o_ref):
    y = jnp.dot(hp_ref[...], w1_ref[...], preferred_element_type=jnp.float32)
    y = jnp.maximum(y + b1_ref[...], 0.0)
    o_ref[...] = jnp.dot(y, w2_ref[...], preferred_element_type=jnp.float32) + b2_ref[...]


def _decode_p(hp, dec_W0, dec_b0, dec_W1, dec_b1):
    f32 = jnp.float32
    e4 = jnp.eye(4, dtype=f32)
    w1p = jnp.kron(e4, dec_W0)
    b1p = jnp.tile(dec_b0, 4)
    w2p = jnp.kron(e4, dec_W1)
    b2p = jnp.tile(dec_b1, 4)
    rb = 256
    out = pl.pallas_call(
        _dec_p_kernel,
        grid=(_NPP // rb,),
        in_specs=[
            pl.BlockSpec((rb, 128), lambda i: (i, 0)),
            _full((128, 256)), _full((1, 256)),
            _full((256, 512)), _full((1, 512)),
        ],
        out_specs=pl.BlockSpec((rb, 512), lambda i: (i, 0)),
        out_shape=jax.ShapeDtypeStruct((_NPP, 512), jnp.float32),
    )(hp, w1p, b1p.reshape(1, -1), w2p, b2p.reshape(1, -1))
    return out.reshape(_NP, _DF)[:_N]


def kernel(nodes, edges, senders, receivers,
           enc_node_W0, enc_node_b0, enc_node_W1, enc_node_b1,
           enc_edge_W0, enc_edge_b0, enc_edge_W1, enc_edge_b1,
           W_message, W_node,
           nodeMLP_W0, nodeMLP_b0, nodeMLP_W1, nodeMLP_b1,
           ln_scale, ln_bias,
           dec_W0, dec_b0, dec_W1, dec_b1):
    senders = senders.astype(jnp.int32)
    receivers = receivers.astype(jnp.int32)
    nodes_p = jnp.pad(nodes, ((0, _NP - _N), (0, 0)))

    f32 = jnp.float32
    wm_top = W_message[:_H]
    wm_bot = W_message[_H:]
    w2c = enc_edge_W1 @ wm_bot
    b2c = enc_edge_b1 @ wm_bot

    e4 = jnp.eye(4, dtype=f32)
    pk = lambda w: jnp.kron(e4, w)
    t4 = lambda b: jnp.tile(b, 4).reshape(1, -1)
    mones = jnp.kron(e4, jnp.full((_H, _H), 1.0 / _H, dtype=f32))
    pw = (pk(wm_top), pk(nodeMLP_W0[:_H]), pk(nodeMLP_W0[_H:]),
          t4(nodeMLP_b0), pk(nodeMLP_W1), t4(nodeMLP_b1), pk(W_node),
          mones, t4(ln_scale), t4(ln_bias))

    h_n = _mlp2(nodes_p, enc_node_W0, enc_node_b0, enc_node_W1, enc_node_b1, 1024)
    hp = h_n.reshape(_NPP, 128)
    s_p = _sc_gather_scatter(h_n, senders, receivers)

    edges_b, hp = lax.optimization_barrier((edges, hp))
    cols = [edges_b[:, k].reshape(_N, _H) for k in range(4)]
    z_e = _edge_encode(cols, enc_edge_W0, enc_edge_b0, w2c, b2c)
    recv_perm = receivers.reshape(10, 1000, 8, 4).transpose(0, 2, 1, 3).reshape(-1)
    agge_p = _sc_segment_sum(z_e, recv_perm)
    aggp_v = agge_p.reshape(_NC, _NPP, 128)

    for step in range(_NMP):
        hp = _update_p(hp, s_p.reshape(_NC, _NPP, 128), aggp_v, pw)
        if step < _NMP - 1:
            s_p = _sc_gather_scatter(hp.reshape(_NP, _H), senders, receivers)

    return _decode_p(hp, dec_W0, dec_b0, dec_W1, dec_b1)
